# baseline (speedup 1.0000x reference)
import functools

import jax
import jax.numpy as jnp
import numpy as np
from jax.experimental import pallas as pl
from jax.experimental.pallas import tpu as pltpu
from jax.sharding import PartitionSpec as P


def _round_up(x, m):
    return (x + m - 1) // m * m


def _tree_max(h):
    r = h.shape[0]
    while r > 8:
        r //= 2
        h = jnp.maximum(h[:r], h[r:2 * r])
    return jnp.max(h, axis=0, keepdims=True)


def _fused_kernel(x_ref, w1t_ref, w2t_ref, b2_ref, w3_ref, b3_ref,
                  w4_ref, b4_ref, w5_ref, b5_ref, out_ref, *, tb):
    w1tv = w1t_ref[...]
    w2tv = w2t_ref[...]
    b2v = b2_ref[...]
    w3v = w3_ref[...]
    w4v = w4_ref[...]
    b4v = b4_ref[...]
    w5v = w5_ref[...]
    b5v = b5_ref[...]

    n = x_ref.shape[2]
    ones_row = jnp.ones((1, n), jnp.bfloat16)
    h2s = []
    for b in range(tb):
        xt4 = jnp.concatenate(
            [x_ref[b].astype(jnp.bfloat16), ones_row], axis=0)
        h = jnp.dot(w1tv, xt4, preferred_element_type=jnp.float32)
        h = jnp.maximum(h, 0.0).astype(jnp.bfloat16)
        h = jnp.dot(w2tv, h, preferred_element_type=jnp.float32) + b2v
        h2s.append(jnp.maximum(h, 0.0).astype(jnp.bfloat16))
    pooled = []
    for b in range(tb):
        h = jax.lax.dot_general(
            h2s[b], w3v, (((0,), (0,)), ((), ())),
            preferred_element_type=jnp.float32)
        pooled.append(_tree_max(h))
    g = pooled[0] if tb == 1 else jnp.concatenate(pooled, axis=0)
    g = jnp.maximum(g + b3_ref[...], 0.0).astype(jnp.bfloat16)

    y = jnp.dot(g, w4v, preferred_element_type=jnp.float32) + b4v
    y = jnp.maximum(y, 0.0).astype(jnp.bfloat16)
    y = jnp.dot(y, w5v, preferred_element_type=jnp.float32) + b5v
    out_ref[...] = jnp.maximum(y, 0.0)


def kernel(state, w1, b1, w2, b2, w3, b3, w4, b4, w5, b5):
    B, C, N = state.shape
    F = w1.shape[1]
    F2 = w2.shape[1]
    H = w5.shape[1]
    F_p = _round_up(F, 128)
    F2_p = _round_up(F2, 128)
    H_p = _round_up(H, 128)

    for t in (64, 32, 16, 8, 4, 2, 1):
        if B % t == 0:
            TB = t
            break

    def pad2(a, rows, cols):
        return jnp.pad(a, ((0, rows - a.shape[0]), (0, cols - a.shape[1])))

    cdt = jnp.bfloat16
    w1tp = pad2(jnp.concatenate([w1, b1], axis=0).T, F_p, C + 1).astype(cdt)
    w2tp = pad2(w2.T, F2_p, F_p).astype(cdt)
    w3p = pad2(w3, F2_p, H_p).astype(cdt)
    w4p = pad2(w4, H_p, H_p).astype(cdt)
    w5p = pad2(w5, H_p, H_p).astype(cdt)
    b2p = pad2(b2.T, F2_p, 1)
    b3p = pad2(b3, 1, H_p)
    b4p = pad2(b4, 1, H_p)
    b5p = pad2(b5, 1, H_p)

    def const_spec(shape):
        return pl.BlockSpec(shape, lambda i, _nd=len(shape): (0,) * _nd)

    def call_pallas(x_local, w1tv, w2tv, b2v, w3v, b3v, w4v, b4v, w5v, b5v):
        b_local = x_local.shape[0]
        return pl.pallas_call(
            functools.partial(_fused_kernel, tb=TB),
            out_shape=jax.ShapeDtypeStruct((b_local, H_p), jnp.float32),
            grid_spec=pltpu.PrefetchScalarGridSpec(
                num_scalar_prefetch=0,
                grid=(b_local // TB,),
                in_specs=[
                    pl.BlockSpec((TB, C, N), lambda i: (i, 0, 0)),
                    const_spec((F_p, C + 1)),
                    const_spec((F2_p, F_p)), const_spec((F2_p, 1)),
                    const_spec((F2_p, H_p)), const_spec((1, H_p)),
                    const_spec((H_p, H_p)), const_spec((1, H_p)),
                    const_spec((H_p, H_p)), const_spec((1, H_p)),
                ],
                out_specs=pl.BlockSpec((TB, H_p), lambda i: (i, 0)),
            ),
            compiler_params=pltpu.CompilerParams(
                dimension_semantics=("parallel",),
                vmem_limit_bytes=64 * 1024 * 1024,
            ),
        )(x_local, w1tv, w2tv, b2v, w3v, b3v, w4v, b4v, w5v, b5v)

    xb = state.astype(jnp.bfloat16)
    weights = (w1tp, w2tp, b2p, w3p, b3p, w4p, b4p, w5p, b5p)

    devs = jax.devices()
    nd = len(devs)
    if nd > 1 and B % (nd * TB) == 0:
        mesh = jax.sharding.Mesh(np.array(devs), ("x",))
        out = jax.shard_map(
            call_pallas, mesh=mesh,
            in_specs=(P("x", None, None),) + (P(None, None),) * len(weights),
            out_specs=P("x", None), check_vma=False,
        )(xb, *weights)
    else:
        out = call_pallas(xb, *weights)

    return out[:, :H] if H_p != H else out


# speedup vs baseline: 1.1110x; 1.1110x over previous
import functools

import jax
import jax.numpy as jnp
import numpy as np
from jax.experimental import pallas as pl
from jax.experimental.pallas import tpu as pltpu
from jax.sharding import PartitionSpec as P


def _round_up(x, m):
    return (x + m - 1) // m * m


def _tree_max(h):
    r = h.shape[0]
    while r > 8:
        r //= 2
        h = jnp.maximum(h[:r], h[r:2 * r])
    return jnp.max(h, axis=0, keepdims=True)


def _fused_kernel(x_ref, w1t_ref, w2t_ref, b2_ref, w3_ref, b3_ref,
                  w4_ref, b4_ref, w5_ref, b5_ref, out_ref, *, tb):
    w1tv = w1t_ref[...]
    w2tv = w2t_ref[...]
    b2v = b2_ref[...]
    w3v = w3_ref[...]
    w4v = w4_ref[...]
    b4v = b4_ref[...]
    w5v = w5_ref[...]
    b5v = b5_ref[...]

    n = x_ref.shape[2]
    ones_row = jnp.ones((1, n), jnp.bfloat16)
    h2s = []
    for b in range(tb):
        xt4 = jnp.concatenate(
            [x_ref[b].astype(jnp.bfloat16), ones_row], axis=0)
        h = jnp.dot(w1tv, xt4, preferred_element_type=jnp.float32)
        h = jnp.maximum(h, 0.0).astype(jnp.bfloat16)
        h = jnp.dot(w2tv, h, preferred_element_type=jnp.float32) + b2v
        h2s.append(jnp.maximum(h, 0.0).astype(jnp.bfloat16))
    pooled = []
    for b in range(tb):
        h = jax.lax.dot_general(
            h2s[b], w3v, (((0,), (0,)), ((), ())),
            preferred_element_type=jnp.float32)
        pooled.append(_tree_max(h))
    g = pooled[0] if tb == 1 else jnp.concatenate(pooled, axis=0)
    g = jnp.maximum(g + b3_ref[...], 0.0).astype(jnp.bfloat16)

    y = jnp.dot(g, w4v, preferred_element_type=jnp.float32) + b4v
    y = jnp.maximum(y, 0.0).astype(jnp.bfloat16)
    y = jnp.dot(y, w5v, preferred_element_type=jnp.float32) + b5v
    out_ref[...] = jnp.maximum(y, 0.0)


def kernel(state, w1, b1, w2, b2, w3, b3, w4, b4, w5, b5):
    B, C, N = state.shape
    F = w1.shape[1]
    F2 = w2.shape[1]
    H = w5.shape[1]
    F_p = _round_up(F, 128)
    F2_p = _round_up(F2, 128)
    H_p = _round_up(H, 128)

    for t in (32, 16, 8, 4, 2, 1):
        if B % t == 0:
            TB = t
            break

    def pad2(a, rows, cols):
        return jnp.pad(a, ((0, rows - a.shape[0]), (0, cols - a.shape[1])))

    cdt = jnp.bfloat16
    w1tp = pad2(jnp.concatenate([w1, b1], axis=0).T, F_p, C + 1).astype(cdt)
    w2tp = pad2(w2.T, F2_p, F_p).astype(cdt)
    w3p = pad2(w3, F2_p, H_p).astype(cdt)
    w4p = pad2(w4, H_p, H_p).astype(cdt)
    w5p = pad2(w5, H_p, H_p).astype(cdt)
    b2p = pad2(b2.T, F2_p, 1)
    b3p = pad2(b3, 1, H_p)
    b4p = pad2(b4, 1, H_p)
    b5p = pad2(b5, 1, H_p)

    def const_spec(shape):
        return pl.BlockSpec(shape, lambda i, _nd=len(shape): (0,) * _nd)

    def call_pallas(x_local, w1tv, w2tv, b2v, w3v, b3v, w4v, b4v, w5v, b5v):
        b_local = x_local.shape[0]
        return pl.pallas_call(
            functools.partial(_fused_kernel, tb=TB),
            out_shape=jax.ShapeDtypeStruct((b_local, H_p), jnp.float32),
            grid_spec=pltpu.PrefetchScalarGridSpec(
                num_scalar_prefetch=0,
                grid=(b_local // TB,),
                in_specs=[
                    pl.BlockSpec((TB, C, N), lambda i: (i, 0, 0)),
                    const_spec((F_p, C + 1)),
                    const_spec((F2_p, F_p)), const_spec((F2_p, 1)),
                    const_spec((F2_p, H_p)), const_spec((1, H_p)),
                    const_spec((H_p, H_p)), const_spec((1, H_p)),
                    const_spec((H_p, H_p)), const_spec((1, H_p)),
                ],
                out_specs=pl.BlockSpec((TB, H_p), lambda i: (i, 0)),
            ),
            compiler_params=pltpu.CompilerParams(
                dimension_semantics=("parallel",),
                vmem_limit_bytes=64 * 1024 * 1024,
            ),
        )(x_local, w1tv, w2tv, b2v, w3v, b3v, w4v, b4v, w5v, b5v)

    xb = state.astype(jnp.bfloat16)
    weights = (w1tp, w2tp, b2p, w3p, b3p, w4p, b4p, w5p, b5p)

    devs = jax.devices()
    nd = len(devs)
    if nd > 1 and B % (nd * TB) == 0:
        mesh = jax.sharding.Mesh(np.array(devs), ("x",))
        out = jax.shard_map(
            call_pallas, mesh=mesh,
            in_specs=(P("x", None, None),) + (P(None, None),) * len(weights),
            out_specs=P("x", None), check_vma=False,
        )(xb, *weights)
    else:
        out = call_pallas(xb, *weights)

    return out[:, :H] if H_p != H else out


# speedup vs baseline: 1.1350x; 1.0216x over previous
import functools

import jax
import jax.numpy as jnp
import numpy as np
from jax.experimental import pallas as pl
from jax.experimental.pallas import tpu as pltpu
from jax.sharding import PartitionSpec as P


def _round_up(x, m):
    return (x + m - 1) // m * m


def _tree_max(h):
    r = h.shape[0]
    while r > 8:
        r //= 2
        h = jnp.maximum(h[:r], h[r:2 * r])
    return jnp.max(h, axis=0, keepdims=True)


def _fused_kernel(x_ref, w1t_ref, w2t_ref, b2_ref, w3_ref, b3_ref,
                  w4_ref, b4_ref, w5_ref, b5_ref, out_ref, *, tb):
    w1tv = w1t_ref[...]
    w2tv = w2t_ref[...]
    b2v = b2_ref[...]
    w3v = w3_ref[...]
    w4v = w4_ref[...]
    b4v = b4_ref[...]
    w5v = w5_ref[...]
    b5v = b5_ref[...]

    n = x_ref.shape[2]
    ones_row = jnp.ones((1, n), jnp.bfloat16)
    h2s = []
    for b in range(tb):
        xt4 = jnp.concatenate(
            [x_ref[b].astype(jnp.bfloat16), ones_row], axis=0)
        h = jnp.dot(w1tv, xt4, preferred_element_type=jnp.float32)
        h = jnp.maximum(h, 0.0).astype(jnp.bfloat16)
        h = jnp.dot(w2tv, h, preferred_element_type=jnp.float32) + b2v
        h2s.append(jnp.maximum(h, 0.0).astype(jnp.bfloat16))
    def dot3(hv):
        return jax.lax.dot_general(
            hv, w3v, (((0,), (0,)), ((), ())),
            preferred_element_type=jnp.float32)
    pooled = []
    h_prev = dot3(h2s[0])
    for b in range(1, tb):
        h_cur = dot3(h2s[b])
        pooled.append(_tree_max(h_prev))
        h_prev = h_cur
    pooled.append(_tree_max(h_prev))
    g = pooled[0] if tb == 1 else jnp.concatenate(pooled, axis=0)
    g = jnp.maximum(g + b3_ref[...], 0.0).astype(jnp.bfloat16)

    y = jnp.dot(g, w4v, preferred_element_type=jnp.float32) + b4v
    y = jnp.maximum(y, 0.0).astype(jnp.bfloat16)
    y = jnp.dot(y, w5v, preferred_element_type=jnp.float32) + b5v
    out_ref[...] = jnp.maximum(y, 0.0)


def kernel(state, w1, b1, w2, b2, w3, b3, w4, b4, w5, b5):
    B, C, N = state.shape
    F = w1.shape[1]
    F2 = w2.shape[1]
    H = w5.shape[1]
    F_p = _round_up(F, 128)
    F2_p = _round_up(F2, 128)
    H_p = _round_up(H, 128)

    for t in (32, 16, 8, 4, 2, 1):
        if B % t == 0:
            TB = t
            break

    def pad2(a, rows, cols):
        return jnp.pad(a, ((0, rows - a.shape[0]), (0, cols - a.shape[1])))

    cdt = jnp.bfloat16
    w1tp = pad2(jnp.concatenate([w1, b1], axis=0).T, F_p, C + 1).astype(cdt)
    w2tp = pad2(w2.T, F2_p, F_p).astype(cdt)
    w3p = pad2(w3, F2_p, H_p).astype(cdt)
    w4p = pad2(w4, H_p, H_p).astype(cdt)
    w5p = pad2(w5, H_p, H_p).astype(cdt)
    b2p = pad2(b2.T, F2_p, 1)
    b3p = pad2(b3, 1, H_p)
    b4p = pad2(b4, 1, H_p)
    b5p = pad2(b5, 1, H_p)

    def const_spec(shape):
        return pl.BlockSpec(shape, lambda i, _nd=len(shape): (0,) * _nd)

    def call_pallas(x_local, w1tv, w2tv, b2v, w3v, b3v, w4v, b4v, w5v, b5v):
        b_local = x_local.shape[0]
        return pl.pallas_call(
            functools.partial(_fused_kernel, tb=TB),
            out_shape=jax.ShapeDtypeStruct((b_local, H_p), jnp.float32),
            grid_spec=pltpu.PrefetchScalarGridSpec(
                num_scalar_prefetch=0,
                grid=(b_local // TB,),
                in_specs=[
                    pl.BlockSpec((TB, C, N), lambda i: (i, 0, 0)),
                    const_spec((F_p, C + 1)),
                    const_spec((F2_p, F_p)), const_spec((F2_p, 1)),
                    const_spec((F2_p, H_p)), const_spec((1, H_p)),
                    const_spec((H_p, H_p)), const_spec((1, H_p)),
                    const_spec((H_p, H_p)), const_spec((1, H_p)),
                ],
                out_specs=pl.BlockSpec((TB, H_p), lambda i: (i, 0)),
            ),
            compiler_params=pltpu.CompilerParams(
                dimension_semantics=("parallel",),
                vmem_limit_bytes=64 * 1024 * 1024,
            ),
        )(x_local, w1tv, w2tv, b2v, w3v, b3v, w4v, b4v, w5v, b5v)

    xb = state.astype(jnp.bfloat16)
    weights = (w1tp, w2tp, b2p, w3p, b3p, w4p, b4p, w5p, b5p)

    devs = jax.devices()
    nd = len(devs)
    if nd > 1 and B % (nd * TB) == 0:
        mesh = jax.sharding.Mesh(np.array(devs), ("x",))
        out = jax.shard_map(
            call_pallas, mesh=mesh,
            in_specs=(P("x", None, None),) + (P(None, None),) * len(weights),
            out_specs=P("x", None), check_vma=False,
        )(xb, *weights)
    else:
        out = call_pallas(xb, *weights)

    return out[:, :H] if H_p != H else out


# speedup vs baseline: 1.1574x; 1.0197x over previous
import functools

import jax
import jax.numpy as jnp
import numpy as np
from jax.experimental import pallas as pl
from jax.experimental.pallas import tpu as pltpu
from jax.sharding import PartitionSpec as P


def _round_up(x, m):
    return (x + m - 1) // m * m


def _tree_max(h):
    r = h.shape[0]
    while r > 8:
        r //= 2
        h = jnp.maximum(h[:r], h[r:2 * r])
    return jnp.max(h, axis=0, keepdims=True)


def _fused_kernel(x_ref, w1t_ref, w2t_ref, b2_ref, w3_ref, b3_ref,
                  w4_ref, b4_ref, w5_ref, b5_ref, out_ref, g_ref, *,
                  tb, groups):
    w1tv = w1t_ref[...]
    w2tv = w2t_ref[...]
    b2v = b2_ref[...]
    w3v = w3_ref[...]
    w4v = w4_ref[...]
    b4v = b4_ref[...]
    w5v = w5_ref[...]
    b5v = b5_ref[...]

    n = x_ref.shape[2]
    gsz = tb // groups
    ones_row = jnp.ones((1, n), jnp.bfloat16)

    def group_body(gi, _):
        base = gi * gsz
        h2s = []
        for b in range(gsz):
            xt4 = jnp.concatenate(
                [x_ref[base + b].astype(jnp.bfloat16), ones_row],
                axis=0)
            h = jnp.dot(w1tv, xt4,
                        preferred_element_type=jnp.float32)
            h = jnp.maximum(h, 0.0).astype(jnp.bfloat16)
            h = jnp.dot(w2tv, h, preferred_element_type=jnp.float32) + b2v
            h2s.append(jnp.maximum(h, 0.0).astype(jnp.bfloat16))
        pooled = []
        for b in range(gsz):
            h = jax.lax.dot_general(
                h2s[b], w3v, (((0,), (0,)), ((), ())),
                preferred_element_type=jnp.float32)
            pooled.append(_tree_max(h))
        g_ref[pl.ds(base, gsz), :] = jnp.concatenate(pooled, axis=0)
        return _

    jax.lax.fori_loop(0, groups, group_body, 0)
    g = jnp.maximum(g_ref[...] + b3_ref[...], 0.0).astype(jnp.bfloat16)

    y = jnp.dot(g, w4v, preferred_element_type=jnp.float32) + b4v
    y = jnp.maximum(y, 0.0).astype(jnp.bfloat16)
    y = jnp.dot(y, w5v, preferred_element_type=jnp.float32) + b5v
    out_ref[...] = jnp.maximum(y, 0.0)


def kernel(state, w1, b1, w2, b2, w3, b3, w4, b4, w5, b5):
    B, C, N = state.shape
    F = w1.shape[1]
    F2 = w2.shape[1]
    H = w5.shape[1]
    F_p = _round_up(F, 128)
    F2_p = _round_up(F2, 128)
    H_p = _round_up(H, 128)

    for t in (64, 32, 16, 8, 4, 2, 1):
        if B % t == 0:
            TB = t
            break
    GROUPS = 2 if TB == 64 else 1

    def pad2(a, rows, cols):
        return jnp.pad(a, ((0, rows - a.shape[0]), (0, cols - a.shape[1])))

    cdt = jnp.bfloat16
    w1tp = pad2(jnp.concatenate([w1, b1], axis=0).T, F_p, C + 1).astype(cdt)
    w2tp = pad2(w2.T, F2_p, F_p).astype(cdt)
    w3p = pad2(w3, F2_p, H_p).astype(cdt)
    w4p = pad2(w4, H_p, H_p).astype(cdt)
    w5p = pad2(w5, H_p, H_p).astype(cdt)
    b2p = pad2(b2.T, F2_p, 1)
    b3p = pad2(b3, 1, H_p)
    b4p = pad2(b4, 1, H_p)
    b5p = pad2(b5, 1, H_p)

    def const_spec(shape):
        return pl.BlockSpec(shape, lambda i, _nd=len(shape): (0,) * _nd)

    def call_pallas(x_local, w1tv, w2tv, b2v, w3v, b3v, w4v, b4v, w5v, b5v):
        b_local = x_local.shape[0]
        return pl.pallas_call(
            functools.partial(_fused_kernel, tb=TB, groups=GROUPS),
            out_shape=jax.ShapeDtypeStruct((b_local, H_p), jnp.float32),
            grid_spec=pltpu.PrefetchScalarGridSpec(
                num_scalar_prefetch=0,
                grid=(b_local // TB,),
                scratch_shapes=[pltpu.VMEM((TB, H_p), jnp.float32)],
                in_specs=[
                    pl.BlockSpec((TB, C, N), lambda i: (i, 0, 0)),
                    const_spec((F_p, C + 1)),
                    const_spec((F2_p, F_p)), const_spec((F2_p, 1)),
                    const_spec((F2_p, H_p)), const_spec((1, H_p)),
                    const_spec((H_p, H_p)), const_spec((1, H_p)),
                    const_spec((H_p, H_p)), const_spec((1, H_p)),
                ],
                out_specs=pl.BlockSpec((TB, H_p), lambda i: (i, 0)),
            ),
            compiler_params=pltpu.CompilerParams(
                dimension_semantics=("parallel",),
                vmem_limit_bytes=64 * 1024 * 1024,
            ),
        )(x_local, w1tv, w2tv, b2v, w3v, b3v, w4v, b4v, w5v, b5v)

    xb = state.astype(jnp.bfloat16)
    weights = (w1tp, w2tp, b2p, w3p, b3p, w4p, b4p, w5p, b5p)

    devs = jax.devices()
    nd = len(devs)
    if nd > 1 and B % (nd * TB) == 0:
        mesh = jax.sharding.Mesh(np.array(devs), ("x",))
        out = jax.shard_map(
            call_pallas, mesh=mesh,
            in_specs=(P("x", None, None),) + (P(None, None),) * len(weights),
            out_specs=P("x", None), check_vma=False,
        )(xb, *weights)
    else:
        out = call_pallas(xb, *weights)

    return out[:, :H] if H_p != H else out
